# baseline (speedup 1.0000x reference)
import functools

import jax
import jax.numpy as jnp
from jax.experimental import pallas as pl

N = 10000
DIM_M = 64
OMEGA = 0.1
K_HYP = 1.0


def _matmul_body(x_ref, w_ref, o_ref):
    o_ref[...] = jnp.dot(x_ref[...], w_ref[...],
                         preferred_element_type=jnp.float32)


def _matmul(x, w, bm=1000):
    m, k = x.shape
    _, n = w.shape
    return pl.pallas_call(
        _matmul_body,
        grid=(m // bm,),
        in_specs=[pl.BlockSpec((bm, k), lambda i: (i, 0)),
                  pl.BlockSpec((k, n), lambda i: (0, 0))],
        out_specs=pl.BlockSpec((bm, n), lambda i: (i, 0)),
        out_shape=jax.ShapeDtypeStruct((m, n), jnp.float32),
    )(x, w)


def kernel(x, adj, degree, idx, edge, W_conv, W_gamma, W_beta, b_gamma,
           b_beta, W_add, W_rev, PE):
    src = adj[0]
    dst = adj[1]
    h = _matmul(x, W_conv)
    loop = jnp.arange(N, dtype=src.dtype)
    src_f = jnp.concatenate([src, loop])
    dst_f = jnp.concatenate([dst, loop])
    deg_sl = jax.ops.segment_sum(jnp.ones(src_f.shape[0], jnp.float32),
                                 dst_f, num_segments=N)
    dis = jnp.where(deg_sl > 0, 1.0 / jnp.sqrt(deg_sl), 0.0)
    normv = dis[src_f] * dis[dst_f]
    conv_out = jax.ops.segment_sum(h[src_f] * normv[:, None], dst_f,
                                   num_segments=N)
    h2 = h * (DIM_M ** 0.5)
    m_dv = jnp.squeeze(PE[degree], axis=1)
    gamma = jax.nn.leaky_relu(m_dv @ W_gamma + b_gamma, negative_slope=0.01)
    beta = jax.nn.leaky_relu(m_dv @ W_beta + b_beta, negative_slope=0.01)
    agg = jax.ops.segment_sum(h2[dst], src, num_segments=N)
    deg_f = degree.astype(jnp.float32)
    deg_safe = jnp.where(deg_f == 0, 1.0, deg_f)
    i_n = agg / deg_safe
    i_n = jnp.where(deg_f == 0, 0.0, i_n)
    ba = _matmul(i_n, jnp.concatenate([W_add, W_rev], axis=1))
    b_add = (gamma + 1.0) * ba[:, :256] + beta
    b_rev = (gamma + 1.0) * ba[:, 256:] + beta
    mean_degree = jnp.mean(deg_f)
    Kv = mean_degree * K_HYP
    R = (deg_f < Kv).astype(jnp.float32)
    nb = float(idx.shape[0])
    L_b = (jnp.sum(jnp.linalg.norm((R * b_add)[idx], axis=1)) +
           jnp.sum(jnp.linalg.norm(((1.0 - R) * b_rev)[idx], axis=1))) / nb
    L_film = (jnp.sum(jnp.linalg.norm(gamma[idx], axis=1)) +
              jnp.sum(jnp.linalg.norm(beta[idx], axis=1))) / nb
    bias = OMEGA * (R * b_add - (1.0 - R) * b_rev)
    output = conv_out + bias
    return output, L_b, L_film


# speedup vs baseline: 5.7953x; 5.7953x over previous
import functools

import jax
import jax.numpy as jnp
from jax import lax
from jax.experimental import pallas as pl
from jax.experimental.pallas import tpu as pltpu
from jax.experimental.pallas import tpu_sc as plsc

N = 10000
E = 160000
C = 256
HC = 128
DIM_M = 64
D_MAX = 612
OMEGA = 0.1
B_IDX = 2048

NC = 2
NS = 16
EB = 128
NB = 80
CH = 4
NCH = NB // CH
ET = EB * NB
E_PAD = ET * NS
JUNK = N
NACC = 10008
NPG = 10240
GB = 5
D_PAD = 616
NBUF = 3

CHUNKS_MAIN = ((0, 128), (128, 128), (256, 128), (384, 128), (512, 120))
CHUNKS_LAST = ((0, 128), (128, 128), (256, 128), (384, 128), (512, 8))
S_LAST = 15


def _stripe_copies(s, copy_fn):
    @pl.when(s < S_LAST)
    def _():
        for off, sz in CHUNKS_MAIN:
            copy_fn(off, sz)

    @pl.when(s == S_LAST)
    def _():
        for off, sz in CHUNKS_LAST:
            copy_fn(off, sz)


def _zero_acc(s, z128, rows_v, acc_sp):
    pltpu.sync_copy(z128, rows_v.at[0])

    def cp(off, sz):
        pltpu.sync_copy(rows_v.at[0, pl.ds(0, sz)],
                        acc_sp.at[pl.ds(s * 632 + off, sz)])
    _stripe_copies(s, cp)


def _write_acc(s, rows_v, acc_sp, out_ref):
    def cp(off, sz):
        pltpu.sync_copy(acc_sp.at[pl.ds(s * 632 + off, sz)],
                        rows_v.at[0, pl.ds(0, sz)])
        pltpu.sync_copy(rows_v.at[0, pl.ds(0, sz)],
                        out_ref.at[pl.ds(s * 632 + off, sz)])
    _stripe_copies(s, cp)


def _ring_edges(table, gidx, sidx, c, s, gi_v, si_v, rows_v, acc_sp,
                sg, ss):
    def chunk_body(ch, _):
        pltpu.sync_copy(gidx.at[c, s, ch], gi_v)
        pltpu.sync_copy(sidx.at[s, ch], si_v)
        g = {}
        sd = {}
        waited = set()
        g[0] = pltpu.async_copy(table.at[gi_v.at[0]], rows_v.at[0], sg[0])
        g[1] = pltpu.async_copy(table.at[gi_v.at[1]], rows_v.at[1], sg[1])
        for b in range(CH):
            rb = b % NBUF
            g[b].wait()
            sd[b] = pltpu.async_copy(rows_v.at[rb], acc_sp.at[si_v.at[b]],
                                     ss[rb], add=True)
            nb = b + 2
            if nb < CH:
                rn = nb % NBUF
                if b >= 1:
                    sd[b - 1].wait()
                    waited.add(b - 1)
                g[nb] = pltpu.async_copy(table.at[gi_v.at[nb]],
                                         rows_v.at[rn], sg[rn])
        for b in range(CH):
            if b not in waited:
                sd[b].wait()
        return _

    lax.fori_loop(0, NCH, chunk_body, None)


def _sc_pass_b_body(tableB, gidxB, sidxB, hidxB, idh, z128, omask,
                    aggB, histH,
                    acc_sp, gi_v, si_v, rows_v, sg0, sg1, sg2,
                    ss0, ss1, ss2):
    c = lax.axis_index("c")
    s = lax.axis_index("s")

    _zero_acc(s, z128, rows_v, acc_sp)
    pltpu.sync_copy(omask.at[0], rows_v.at[0])
    pltpu.sync_copy(omask.at[1], rows_v.at[1])
    plsc.subcore_barrier()

    ones0 = rows_v.at[0]
    HCH = NCH // 2

    def hchunk_body(ch, _):
        pltpu.sync_copy(hidxB.at[s, ch + c * HCH], gi_v)
        descs = [pltpu.async_copy(ones0, acc_sp.at[gi_v.at[b]], sg0,
                                  add=True) for b in range(CH)]
        for d in descs:
            d.wait()
        return _
    lax.fori_loop(0, HCH, hchunk_body, None)

    @pl.when(c == 1)
    def _():
        pltpu.sync_copy(idh.at[s], gi_v.at[pl.ds(0, 1)])
        pltpu.sync_copy(rows_v.at[1], acc_sp.at[gi_v.at[0]], add=True)

    plsc.subcore_barrier()
    _write_acc(s, rows_v, acc_sp, histH.at[c])
    _zero_acc(s, z128, rows_v, acc_sp)
    plsc.subcore_barrier()

    _ring_edges(tableB, gidxB, sidxB, c, s, gi_v, si_v, rows_v, acc_sp,
                (sg0, sg1, sg2), (ss0, ss1, ss2))

    plsc.subcore_barrier()
    _write_acc(s, rows_v, acc_sp, aggB.at[c])


def _sc_pass_a_body(tableA, gidxA, sidxA, dgidx, tableG, tableBt, z128,
                    aggA, gammaM, betaM,
                    acc_sp, gi_v, si_v, rows_v, sg0, sg1, sg2,
                    ss0, ss1, ss2):
    c = lax.axis_index("c")
    s = lax.axis_index("s")
    _zero_acc(s, z128, rows_v, acc_sp)
    plsc.subcore_barrier()

    _ring_edges(tableA, gidxA, sidxA, c, s, gi_v, si_v, rows_v, acc_sp,
                (sg0, sg1, sg2), (ss0, ss1, ss2))

    pltpu.sync_copy(dgidx.at[c, s, pl.ds(0, 4)], gi_v)
    pltpu.sync_copy(dgidx.at[c, s, pl.ds(4, 1)], si_v.at[pl.ds(0, 1)])

    for gb in range(GB):
        ix = gi_v.at[gb] if gb < 4 else si_v.at[0]
        dg = pltpu.async_copy(tableG.at[ix], rows_v.at[0], sg0)
        db = pltpu.async_copy(tableBt.at[ix], rows_v.at[1], sg1)
        dg.wait()
        pltpu.sync_copy(rows_v.at[0],
                        gammaM.at[c, pl.ds(s * 640 + gb * EB, EB)])
        db.wait()
        pltpu.sync_copy(rows_v.at[1],
                        betaM.at[c, pl.ds(s * 640 + gb * EB, EB)])

    plsc.subcore_barrier()
    _write_acc(s, rows_v, acc_sp, aggA.at[c])


_SC_MESH = plsc.VectorSubcoreMesh(core_axis_name="c", subcore_axis_name="s")

_SC_SCRATCH = [
    pltpu.VMEM_SHARED((NACC, HC), jnp.float32),
    pltpu.VMEM((CH, EB), jnp.int32),
    pltpu.VMEM((CH, EB), jnp.int32),
    pltpu.VMEM((NBUF, EB, HC), jnp.float32),
    pltpu.SemaphoreType.DMA,
    pltpu.SemaphoreType.DMA,
    pltpu.SemaphoreType.DMA,
    pltpu.SemaphoreType.DMA,
    pltpu.SemaphoreType.DMA,
    pltpu.SemaphoreType.DMA,
]

_sc_pass_b = functools.partial(
    pl.kernel,
    _sc_pass_b_body,
    out_type=[
        jax.ShapeDtypeStruct((NC, NACC, HC), jnp.float32),
        jax.ShapeDtypeStruct((NC, NACC, HC), jnp.float32),
    ],
    mesh=_SC_MESH,
    scratch_types=_SC_SCRATCH,
)()

_sc_pass_a = functools.partial(
    pl.kernel,
    _sc_pass_a_body,
    out_type=[
        jax.ShapeDtypeStruct((NC, NACC, HC), jnp.float32),
        jax.ShapeDtypeStruct((NC, NPG, HC), jnp.float32),
        jax.ShapeDtypeStruct((NC, NPG, HC), jnp.float32),
    ],
    mesh=_SC_MESH,
    scratch_types=_SC_SCRATCH,
)()


BM = 1000


def _prep_body(x_ref, w_ref, pe_ref, wg_ref, wb_ref, bg_ref, bb_ref,
               degp_ref, h_ref, g_ref, bt_ref, sd_ref):
    i = pl.program_id(0)
    h_ref[...] = jnp.dot(x_ref[...], w_ref[...],
                         preferred_element_type=jnp.float32)

    @pl.when(i == 0)
    def _():
        g_ref[...] = jnp.dot(pe_ref[...], wg_ref[...],
                             preferred_element_type=jnp.float32) + bg_ref[...]
        bt_ref[...] = jnp.dot(pe_ref[...], wb_ref[...],
                              preferred_element_type=jnp.float32) + bb_ref[...]
        sd_ref[...] = jnp.sum(degp_ref[...]).reshape(1, 1)


def _mid_body(histH_ref, h_ref, hs_ref):
    deg_sl = histH_ref[0, :, 0:1] + histH_ref[1, :, 0:1] + 1.0
    hs_ref[...] = h_ref[...] * lax.rsqrt(deg_sl)


def _leaky(v):
    return jnp.where(v >= 0.0, v, 0.01 * v)


def _final_body(aggA_ref, aggB_ref, h_ref, histH_ref,
                gM_ref, bM_ref, deg_ref, wcat_ref, sd_ref,
                out_ref, lb_ref, lf_ref):
    i = pl.program_id(0)
    gamma = _leaky(jnp.concatenate([gM_ref[0], gM_ref[1]], axis=-1))
    beta = _leaky(jnp.concatenate([bM_ref[0], bM_ref[1]], axis=-1))
    deg = deg_ref[...]
    dis = lax.rsqrt(histH_ref[0, :, 0:1] + histH_ref[1, :, 0:1] + 1.0)
    A = jnp.concatenate([aggA_ref[0], aggA_ref[1]], axis=-1)
    conv_out = dis * A + (dis * dis) * h_ref[...]
    B8 = jnp.concatenate([aggB_ref[0], aggB_ref[1]], axis=-1) * 8.0
    zerod = deg == 0.0
    i_n = jnp.where(zerod, 0.0, B8 / jnp.where(zerod, 1.0, deg))
    t = jnp.dot(i_n, wcat_ref[...], preferred_element_type=jnp.float32)
    b_add = (gamma + 1.0) * t[:, :C] + beta
    b_rev = (gamma + 1.0) * t[:, C:] + beta
    Kv = sd_ref[...][0, 0] * (1.0 / N)
    R = (deg < Kv).astype(jnp.float32)
    out_ref[...] = conv_out + OMEGA * (R * b_add - (1.0 - R) * b_rev)
    na = jnp.sqrt(jnp.sum(b_add * b_add, axis=1, keepdims=True))
    nr = jnp.sqrt(jnp.sum(b_rev * b_rev, axis=1, keepdims=True))
    ng = jnp.sqrt(jnp.sum(gamma * gamma, axis=1, keepdims=True))
    nbt = jnp.sqrt(jnp.sum(beta * beta, axis=1, keepdims=True))
    cvec = histH_ref[0, :, 64:65] + histH_ref[1, :, 64:65]
    lb_p = jnp.sum(cvec * (R * na + (1.0 - R) * nr)) * (1.0 / B_IDX)
    lf_p = jnp.sum(cvec * (ng + nbt)) * (1.0 / B_IDX)

    @pl.when(i == 0)
    def _():
        lb_ref[...] = jnp.zeros((1, 1), jnp.float32)
        lf_ref[...] = jnp.zeros((1, 1), jnp.float32)

    lb_ref[...] += lb_p.reshape(1, 1)
    lf_ref[...] += lf_p.reshape(1, 1)


def kernel(x, adj, degree, idx, edge, W_conv, W_gamma, W_beta, b_gamma,
           b_beta, W_add, W_rev, PE):
    src = adj[0]
    dst = adj[1]
    f32 = jnp.float32

    padn = E_PAD - E
    srcp = jnp.concatenate([src, jnp.full((padn,), JUNK, jnp.int32)])
    dstp_s = jnp.concatenate([dst, jnp.full((padn,), JUNK, jnp.int32)])
    src0 = jnp.concatenate([src, jnp.zeros((padn,), jnp.int32)])
    dst0 = jnp.concatenate([dst, jnp.zeros((padn,), jnp.int32)])
    gidxB = jnp.stack([2 * dst0, 2 * dst0 + 1]).reshape(NC, NS, NCH, CH, EB)
    sidxB = srcp.reshape(NS, NCH, CH, EB)
    hidxB = dstp_s.reshape(NS, NCH, CH, EB)
    gidxA = jnp.stack([2 * src0, 2 * src0 + 1]).reshape(NC, NS, NCH, CH, EB)
    sidxA = dstp_s.reshape(NS, NCH, CH, EB)
    idh = idx.reshape(NS, 1, EB)
    deg0 = jnp.concatenate([degree[:, 0], jnp.zeros((NPG - N,), jnp.int32)])
    dgidx = jnp.stack([2 * deg0, 2 * deg0 + 1]).reshape(NC, NS, GB, EB)
    z128 = jnp.zeros((EB, HC), f32)
    col = jnp.arange(HC)[None, :]
    omask = jnp.stack([jnp.where(col < 64, 1.0, 0.0) * jnp.ones((EB, 1)),
                       jnp.where(col >= 64, 1.0, 0.0) * jnp.ones((EB, 1))],
                      axis=0).astype(f32)
    deg_f = degree.astype(f32)
    degp = jnp.concatenate([deg_f[:, 0], jnp.zeros((NPG - N,), f32)])
    degp = degp.reshape(80, 128)
    PEp = jnp.concatenate([PE, jnp.zeros((D_PAD - D_MAX, DIM_M), f32)])

    h, G, Bt, sd = pl.pallas_call(
        _prep_body,
        grid=(N // BM,),
        in_specs=[
            pl.BlockSpec((BM, C), lambda i: (i, 0)),
            pl.BlockSpec((C, C), lambda i: (0, 0)),
            pl.BlockSpec((D_PAD, DIM_M), lambda i: (0, 0)),
            pl.BlockSpec((DIM_M, C), lambda i: (0, 0)),
            pl.BlockSpec((DIM_M, C), lambda i: (0, 0)),
            pl.BlockSpec((1, C), lambda i: (0, 0)),
            pl.BlockSpec((1, C), lambda i: (0, 0)),
            pl.BlockSpec((80, 128), lambda i: (0, 0)),
        ],
        out_specs=[
            pl.BlockSpec((BM, C), lambda i: (i, 0)),
            pl.BlockSpec((D_PAD, C), lambda i: (0, 0)),
            pl.BlockSpec((D_PAD, C), lambda i: (0, 0)),
            pl.BlockSpec((1, 1), lambda i: (0, 0)),
        ],
        out_shape=[
            jax.ShapeDtypeStruct((N, C), f32),
            jax.ShapeDtypeStruct((D_PAD, C), f32),
            jax.ShapeDtypeStruct((D_PAD, C), f32),
            jax.ShapeDtypeStruct((1, 1), f32),
        ],
    )(x, W_conv, PEp, W_gamma, W_beta, b_gamma, b_beta, degp)

    tableB = h.reshape(2 * N, HC)
    aggB, histH = _sc_pass_b(tableB, gidxB, sidxB, hidxB, idh, z128, omask)

    hs = pl.pallas_call(
        _mid_body,
        grid=(N // BM,),
        in_specs=[
            pl.BlockSpec((NC, BM, HC), lambda i: (0, i, 0)),
            pl.BlockSpec((BM, C), lambda i: (i, 0)),
        ],
        out_specs=pl.BlockSpec((BM, C), lambda i: (i, 0)),
        out_shape=jax.ShapeDtypeStruct((N, C), f32),
    )(histH, h)

    tableA = hs.reshape(2 * N, HC)
    tableG = G.reshape(2 * D_PAD, HC)
    tableBt = Bt.reshape(2 * D_PAD, HC)
    aggA, gammaM, betaM = _sc_pass_a(tableA, gidxA, sidxA, dgidx,
                                     tableG, tableBt, z128)

    Wcat = jnp.concatenate([W_add, W_rev], axis=1)
    out, lb, lf = pl.pallas_call(
        _final_body,
        grid=(N // BM,),
        in_specs=[
            pl.BlockSpec((NC, BM, HC), lambda i: (0, i, 0)),
            pl.BlockSpec((NC, BM, HC), lambda i: (0, i, 0)),
            pl.BlockSpec((BM, C), lambda i: (i, 0)),
            pl.BlockSpec((NC, BM, HC), lambda i: (0, i, 0)),
            pl.BlockSpec((NC, BM, HC), lambda i: (0, i, 0)),
            pl.BlockSpec((NC, BM, HC), lambda i: (0, i, 0)),
            pl.BlockSpec((BM, 1), lambda i: (i, 0)),
            pl.BlockSpec((C, 2 * C), lambda i: (0, 0)),
            pl.BlockSpec((1, 1), lambda i: (0, 0)),
        ],
        out_specs=[
            pl.BlockSpec((BM, C), lambda i: (i, 0)),
            pl.BlockSpec((1, 1), lambda i: (0, 0)),
            pl.BlockSpec((1, 1), lambda i: (0, 0)),
        ],
        out_shape=[
            jax.ShapeDtypeStruct((N, C), f32),
            jax.ShapeDtypeStruct((1, 1), f32),
            jax.ShapeDtypeStruct((1, 1), f32),
        ],
    )(aggA, aggB, h, histH, gammaM, betaM, deg_f, Wcat, sd)

    return out, lb[0, 0], lf[0, 0]


# speedup vs baseline: 6.0485x; 1.0437x over previous
import functools

import jax
import jax.numpy as jnp
from jax import lax
from jax.experimental import pallas as pl
from jax.experimental.pallas import tpu as pltpu
from jax.experimental.pallas import tpu_sc as plsc

N = 10000
E = 160000
C = 256
HC = 128
DIM_M = 64
D_MAX = 612
OMEGA = 0.1
B_IDX = 2048

NC = 2
NS = 16
EB = 128
NB = 80
CH = 8
NCH = NB // CH
ET = EB * NB
E_PAD = ET * NS
JUNK = N
NACC = 10008
NPG = 10240
GB = 5
D_PAD = 616
NBUF = 2

CHUNKS_MAIN = ((0, 128), (128, 128), (256, 128), (384, 128), (512, 120))
CHUNKS_LAST = ((0, 128), (128, 128), (256, 128), (384, 128), (512, 8))
S_LAST = 15


def _stripe_copies(s, copy_fn):
    @pl.when(s < S_LAST)
    def _():
        for off, sz in CHUNKS_MAIN:
            copy_fn(off, sz)

    @pl.when(s == S_LAST)
    def _():
        for off, sz in CHUNKS_LAST:
            copy_fn(off, sz)


def _zero_acc(s, z128, rows_v, acc_sp):
    pltpu.sync_copy(z128, rows_v.at[0])

    def cp(off, sz):
        pltpu.sync_copy(rows_v.at[0, pl.ds(0, sz)],
                        acc_sp.at[pl.ds(s * 632 + off, sz)])
    _stripe_copies(s, cp)


def _write_acc(s, rows_v, acc_sp, out_ref):
    def cp(off, sz):
        pltpu.sync_copy(acc_sp.at[pl.ds(s * 632 + off, sz)],
                        rows_v.at[0, pl.ds(0, sz)])
        pltpu.sync_copy(rows_v.at[0, pl.ds(0, sz)],
                        out_ref.at[pl.ds(s * 632 + off, sz)])
    _stripe_copies(s, cp)


def _ring_edges(table, gidx, sidx, c, s, gi_v, si_v, rows_v, acc_sp,
                sg, ss):
    def chunk_body(ch, _):
        pltpu.sync_copy(gidx.at[c, s, ch], gi_v)
        pltpu.sync_copy(sidx.at[s, ch], si_v)
        d = pltpu.async_copy(table.at[gi_v.at[0]], rows_v.at[0], sg[0])
        for b in range(CH):
            p = b % 2
            if b + 1 < CH:
                d_next = pltpu.async_copy(table.at[gi_v.at[b + 1]],
                                          rows_v.at[1 - p], sg[1 - p])
            d.wait()
            pltpu.sync_copy(rows_v.at[p], acc_sp.at[si_v.at[b]], add=True)
            if b + 1 < CH:
                d = d_next
        return _

    lax.fori_loop(0, NCH, chunk_body, None)


def _sc_pass_b_body(tableB, gidxB, sidxB, hidxB, idh, z128, omask,
                    aggB, histH,
                    acc_sp, gi_v, si_v, rows_v, sg0, sg1, sg2,
                    ss0, ss1, ss2):
    c = lax.axis_index("c")
    s = lax.axis_index("s")

    _zero_acc(s, z128, rows_v, acc_sp)
    pltpu.sync_copy(omask.at[0], rows_v.at[0])
    pltpu.sync_copy(omask.at[1], rows_v.at[1])
    plsc.subcore_barrier()

    ones0 = rows_v.at[0]
    HCH = NCH // 2

    def hchunk_body(ch, _):
        pltpu.sync_copy(hidxB.at[s, ch + c * HCH], gi_v)
        descs = [pltpu.async_copy(ones0, acc_sp.at[gi_v.at[b]], sg0,
                                  add=True) for b in range(CH)]
        for d in descs:
            d.wait()
        return _
    lax.fori_loop(0, HCH, hchunk_body, None)

    @pl.when(c == 1)
    def _():
        pltpu.sync_copy(idh.at[s], gi_v.at[pl.ds(0, 1)])
        pltpu.sync_copy(rows_v.at[1], acc_sp.at[gi_v.at[0]], add=True)

    plsc.subcore_barrier()
    _write_acc(s, rows_v, acc_sp, histH.at[c])
    _zero_acc(s, z128, rows_v, acc_sp)
    plsc.subcore_barrier()

    _ring_edges(tableB, gidxB, sidxB, c, s, gi_v, si_v, rows_v, acc_sp,
                (sg0, sg1, sg2), (ss0, ss1, ss2))

    plsc.subcore_barrier()
    _write_acc(s, rows_v, acc_sp, aggB.at[c])


def _sc_pass_a_body(tableA, gidxA, sidxA, dgidx, tableG, tableBt, z128,
                    aggA, gammaM, betaM,
                    acc_sp, gi_v, si_v, rows_v, sg0, sg1, sg2,
                    ss0, ss1, ss2):
    c = lax.axis_index("c")
    s = lax.axis_index("s")
    _zero_acc(s, z128, rows_v, acc_sp)
    plsc.subcore_barrier()

    _ring_edges(tableA, gidxA, sidxA, c, s, gi_v, si_v, rows_v, acc_sp,
                (sg0, sg1, sg2), (ss0, ss1, ss2))

    pltpu.sync_copy(dgidx.at[c, s], gi_v.at[pl.ds(0, GB)])

    for gb in range(GB):
        ix = gi_v.at[gb]
        dg = pltpu.async_copy(tableG.at[ix], rows_v.at[0], sg0)
        db = pltpu.async_copy(tableBt.at[ix], rows_v.at[1], sg1)
        dg.wait()
        pltpu.sync_copy(rows_v.at[0],
                        gammaM.at[c, pl.ds(s * 640 + gb * EB, EB)])
        db.wait()
        pltpu.sync_copy(rows_v.at[1],
                        betaM.at[c, pl.ds(s * 640 + gb * EB, EB)])

    plsc.subcore_barrier()
    _write_acc(s, rows_v, acc_sp, aggA.at[c])


_SC_MESH = plsc.VectorSubcoreMesh(core_axis_name="c", subcore_axis_name="s")

_SC_SCRATCH = [
    pltpu.VMEM_SHARED((NACC, HC), jnp.float32),
    pltpu.VMEM((CH, EB), jnp.int32),
    pltpu.VMEM((CH, EB), jnp.int32),
    pltpu.VMEM((NBUF, EB, HC), jnp.float32),
    pltpu.SemaphoreType.DMA,
    pltpu.SemaphoreType.DMA,
    pltpu.SemaphoreType.DMA,
    pltpu.SemaphoreType.DMA,
    pltpu.SemaphoreType.DMA,
    pltpu.SemaphoreType.DMA,
]

_sc_pass_b = functools.partial(
    pl.kernel,
    _sc_pass_b_body,
    out_type=[
        jax.ShapeDtypeStruct((NC, NACC, HC), jnp.float32),
        jax.ShapeDtypeStruct((NC, NACC, HC), jnp.float32),
    ],
    mesh=_SC_MESH,
    scratch_types=_SC_SCRATCH,
)()

_sc_pass_a = functools.partial(
    pl.kernel,
    _sc_pass_a_body,
    out_type=[
        jax.ShapeDtypeStruct((NC, NACC, HC), jnp.float32),
        jax.ShapeDtypeStruct((NC, NPG, HC), jnp.float32),
        jax.ShapeDtypeStruct((NC, NPG, HC), jnp.float32),
    ],
    mesh=_SC_MESH,
    scratch_types=_SC_SCRATCH,
)()


BM = 1000


def _prep_body(x_ref, w_ref, pe_ref, wg_ref, wb_ref, bg_ref, bb_ref,
               degp_ref, h_ref, g_ref, bt_ref, sd_ref):
    i = pl.program_id(0)
    h_ref[...] = jnp.dot(x_ref[...], w_ref[...],
                         preferred_element_type=jnp.float32)

    @pl.when(i == 0)
    def _():
        g_ref[...] = jnp.dot(pe_ref[...], wg_ref[...],
                             preferred_element_type=jnp.float32) + bg_ref[...]
        bt_ref[...] = jnp.dot(pe_ref[...], wb_ref[...],
                              preferred_element_type=jnp.float32) + bb_ref[...]
        sd_ref[...] = jnp.sum(degp_ref[...]).reshape(1, 1)


def _mid_body(histH_ref, h_ref, hs_ref):
    deg_sl = histH_ref[0, :, 0:1] + histH_ref[1, :, 0:1] + 1.0
    hs_ref[...] = h_ref[...] * lax.rsqrt(deg_sl)


def _leaky(v):
    return jnp.where(v >= 0.0, v, 0.01 * v)


def _final_body(aggA_ref, aggB_ref, h_ref, histH_ref,
                gM_ref, bM_ref, deg_ref, wcat_ref, sd_ref,
                out_ref, lb_ref, lf_ref):
    i = pl.program_id(0)
    gamma = _leaky(jnp.concatenate([gM_ref[0], gM_ref[1]], axis=-1))
    beta = _leaky(jnp.concatenate([bM_ref[0], bM_ref[1]], axis=-1))
    deg = deg_ref[...]
    dis = lax.rsqrt(histH_ref[0, :, 0:1] + histH_ref[1, :, 0:1] + 1.0)
    A = jnp.concatenate([aggA_ref[0], aggA_ref[1]], axis=-1)
    conv_out = dis * A + (dis * dis) * h_ref[...]
    B8 = jnp.concatenate([aggB_ref[0], aggB_ref[1]], axis=-1) * 8.0
    zerod = deg == 0.0
    i_n = jnp.where(zerod, 0.0, B8 / jnp.where(zerod, 1.0, deg))
    t = jnp.dot(i_n, wcat_ref[...], preferred_element_type=jnp.float32)
    b_add = (gamma + 1.0) * t[:, :C] + beta
    b_rev = (gamma + 1.0) * t[:, C:] + beta
    Kv = sd_ref[...][0, 0] * (1.0 / N)
    R = (deg < Kv).astype(jnp.float32)
    out_ref[...] = conv_out + OMEGA * (R * b_add - (1.0 - R) * b_rev)
    na = jnp.sqrt(jnp.sum(b_add * b_add, axis=1, keepdims=True))
    nr = jnp.sqrt(jnp.sum(b_rev * b_rev, axis=1, keepdims=True))
    ng = jnp.sqrt(jnp.sum(gamma * gamma, axis=1, keepdims=True))
    nbt = jnp.sqrt(jnp.sum(beta * beta, axis=1, keepdims=True))
    cvec = histH_ref[0, :, 64:65] + histH_ref[1, :, 64:65]
    lb_p = jnp.sum(cvec * (R * na + (1.0 - R) * nr)) * (1.0 / B_IDX)
    lf_p = jnp.sum(cvec * (ng + nbt)) * (1.0 / B_IDX)

    @pl.when(i == 0)
    def _():
        lb_ref[...] = jnp.zeros((1, 1), jnp.float32)
        lf_ref[...] = jnp.zeros((1, 1), jnp.float32)

    lb_ref[...] += lb_p.reshape(1, 1)
    lf_ref[...] += lf_p.reshape(1, 1)


def kernel(x, adj, degree, idx, edge, W_conv, W_gamma, W_beta, b_gamma,
           b_beta, W_add, W_rev, PE):
    src = adj[0]
    dst = adj[1]
    f32 = jnp.float32

    padn = E_PAD - E
    srcp = jnp.concatenate([src, jnp.full((padn,), JUNK, jnp.int32)])
    dstp_s = jnp.concatenate([dst, jnp.full((padn,), JUNK, jnp.int32)])
    src0 = jnp.concatenate([src, jnp.zeros((padn,), jnp.int32)])
    dst0 = jnp.concatenate([dst, jnp.zeros((padn,), jnp.int32)])
    gidxB = jnp.stack([2 * dst0, 2 * dst0 + 1]).reshape(NC, NS, NCH, CH, EB)
    sidxB = srcp.reshape(NS, NCH, CH, EB)
    hidxB = dstp_s.reshape(NS, NCH, CH, EB)
    gidxA = jnp.stack([2 * src0, 2 * src0 + 1]).reshape(NC, NS, NCH, CH, EB)
    sidxA = dstp_s.reshape(NS, NCH, CH, EB)
    idh = idx.reshape(NS, 1, EB)
    deg0 = jnp.concatenate([degree[:, 0], jnp.zeros((NPG - N,), jnp.int32)])
    dgidx = jnp.stack([2 * deg0, 2 * deg0 + 1]).reshape(NC, NS, GB, EB)
    z128 = jnp.zeros((EB, HC), f32)
    col = jnp.arange(HC)[None, :]
    omask = jnp.stack([jnp.where(col < 64, 1.0, 0.0) * jnp.ones((EB, 1)),
                       jnp.where(col >= 64, 1.0, 0.0) * jnp.ones((EB, 1))],
                      axis=0).astype(f32)
    deg_f = degree.astype(f32)
    degp = jnp.concatenate([deg_f[:, 0], jnp.zeros((NPG - N,), f32)])
    degp = degp.reshape(80, 128)
    PEp = jnp.concatenate([PE, jnp.zeros((D_PAD - D_MAX, DIM_M), f32)])

    h, G, Bt, sd = pl.pallas_call(
        _prep_body,
        grid=(N // BM,),
        in_specs=[
            pl.BlockSpec((BM, C), lambda i: (i, 0)),
            pl.BlockSpec((C, C), lambda i: (0, 0)),
            pl.BlockSpec((D_PAD, DIM_M), lambda i: (0, 0)),
            pl.BlockSpec((DIM_M, C), lambda i: (0, 0)),
            pl.BlockSpec((DIM_M, C), lambda i: (0, 0)),
            pl.BlockSpec((1, C), lambda i: (0, 0)),
            pl.BlockSpec((1, C), lambda i: (0, 0)),
            pl.BlockSpec((80, 128), lambda i: (0, 0)),
        ],
        out_specs=[
            pl.BlockSpec((BM, C), lambda i: (i, 0)),
            pl.BlockSpec((D_PAD, C), lambda i: (0, 0)),
            pl.BlockSpec((D_PAD, C), lambda i: (0, 0)),
            pl.BlockSpec((1, 1), lambda i: (0, 0)),
        ],
        out_shape=[
            jax.ShapeDtypeStruct((N, C), f32),
            jax.ShapeDtypeStruct((D_PAD, C), f32),
            jax.ShapeDtypeStruct((D_PAD, C), f32),
            jax.ShapeDtypeStruct((1, 1), f32),
        ],
    )(x, W_conv, PEp, W_gamma, W_beta, b_gamma, b_beta, degp)

    tableB = h.reshape(2 * N, HC)
    aggB, histH = _sc_pass_b(tableB, gidxB, sidxB, hidxB, idh, z128, omask)

    hs = pl.pallas_call(
        _mid_body,
        grid=(N // BM,),
        in_specs=[
            pl.BlockSpec((NC, BM, HC), lambda i: (0, i, 0)),
            pl.BlockSpec((BM, C), lambda i: (i, 0)),
        ],
        out_specs=pl.BlockSpec((BM, C), lambda i: (i, 0)),
        out_shape=jax.ShapeDtypeStruct((N, C), f32),
    )(histH, h)

    tableA = hs.reshape(2 * N, HC)
    tableG = G.reshape(2 * D_PAD, HC)
    tableBt = Bt.reshape(2 * D_PAD, HC)
    aggA, gammaM, betaM = _sc_pass_a(tableA, gidxA, sidxA, dgidx,
                                     tableG, tableBt, z128)

    Wcat = jnp.concatenate([W_add, W_rev], axis=1)
    out, lb, lf = pl.pallas_call(
        _final_body,
        grid=(N // BM,),
        in_specs=[
            pl.BlockSpec((NC, BM, HC), lambda i: (0, i, 0)),
            pl.BlockSpec((NC, BM, HC), lambda i: (0, i, 0)),
            pl.BlockSpec((BM, C), lambda i: (i, 0)),
            pl.BlockSpec((NC, BM, HC), lambda i: (0, i, 0)),
            pl.BlockSpec((NC, BM, HC), lambda i: (0, i, 0)),
            pl.BlockSpec((NC, BM, HC), lambda i: (0, i, 0)),
            pl.BlockSpec((BM, 1), lambda i: (i, 0)),
            pl.BlockSpec((C, 2 * C), lambda i: (0, 0)),
            pl.BlockSpec((1, 1), lambda i: (0, 0)),
        ],
        out_specs=[
            pl.BlockSpec((BM, C), lambda i: (i, 0)),
            pl.BlockSpec((1, 1), lambda i: (0, 0)),
            pl.BlockSpec((1, 1), lambda i: (0, 0)),
        ],
        out_shape=[
            jax.ShapeDtypeStruct((N, C), f32),
            jax.ShapeDtypeStruct((1, 1), f32),
            jax.ShapeDtypeStruct((1, 1), f32),
        ],
    )(aggA, aggB, h, histH, gammaM, betaM, deg_f, Wcat, sd)

    return out, lb[0, 0], lf[0, 0]


# speedup vs baseline: 6.4403x; 1.0648x over previous
import functools

import jax
import jax.numpy as jnp
from jax import lax
from jax.experimental import pallas as pl
from jax.experimental.pallas import tpu as pltpu
from jax.experimental.pallas import tpu_sc as plsc

N = 10000
E = 160000
C = 256
HC = 128
DIM_M = 64
D_MAX = 612
OMEGA = 0.1
B_IDX = 2048

NC = 2
NS = 16
EB = 128
NB = 80
CH = 8
NCH = NB // CH
ET = EB * NB
E_PAD = ET * NS
JUNK = N
NACC = 10112
NPG = 10240
GB = 5
D_PAD = 616
NBUF = 2

CHUNKS_MAIN = ((0, 128), (128, 128), (256, 128), (384, 128), (512, 120))


def _stripe_copies(s, copy_fn):
    for off, sz in CHUNKS_MAIN:
        copy_fn(off, sz)


def _zero_acc(s, z128, rows_v, acc_sp):
    pltpu.sync_copy(z128, rows_v.at[0])

    def cp(off, sz):
        pltpu.sync_copy(rows_v.at[0, pl.ds(0, sz)],
                        acc_sp.at[pl.ds(s * 632 + off, sz)])
    _stripe_copies(s, cp)


def _write_acc(s, rows_v, acc_sp, out_ref):
    def cp(off, sz):
        pltpu.sync_copy(acc_sp.at[pl.ds(s * 632 + off, sz)],
                        rows_v.at[0, pl.ds(0, sz)])
        pltpu.sync_copy(rows_v.at[0, pl.ds(0, sz)],
                        out_ref.at[pl.ds(s * 632 + off, sz)])
    _stripe_copies(s, cp)


def _ring_edges(table, gidx, sidx, c, s, gi_v, si_v, rows_v, acc_sp,
                sg):
    def chunk_body(ch, _):
        pltpu.sync_copy(gidx.at[c, s, ch], gi_v)
        pltpu.sync_copy(sidx.at[s, ch], si_v)
        d = pltpu.async_copy(table.at[gi_v.at[0]], rows_v.at[0], sg[0])
        for b in range(CH):
            p = b % 2
            if b + 1 < CH:
                d_next = pltpu.async_copy(table.at[gi_v.at[b + 1]],
                                          rows_v.at[1 - p], sg[1 - p])
            d.wait()
            pltpu.sync_copy(rows_v.at[p], acc_sp.at[si_v.at[b]], add=True)
            if b + 1 < CH:
                d = d_next
        return _

    lax.fori_loop(0, NCH, chunk_body, None)


def _sc_pass_b_body(tableB, gidxB, sidxB, hidxB, idh, z128, omask,
                    aggB, histH,
                    acc_sp, gi_v, si_v, rows_v, sg0, sg1):
    c = lax.axis_index("c")
    s = lax.axis_index("s")

    _zero_acc(s, z128, rows_v, acc_sp)
    pltpu.sync_copy(omask.at[0], rows_v.at[0])
    pltpu.sync_copy(omask.at[1], rows_v.at[1])
    plsc.subcore_barrier()

    ones0 = rows_v.at[0]
    HCH = NCH // 2

    def hchunk_body(ch, _):
        pltpu.sync_copy(hidxB.at[s, ch + c * HCH], gi_v)
        descs = [pltpu.async_copy(ones0, acc_sp.at[gi_v.at[b]], sg0,
                                  add=True) for b in range(CH)]
        for d in descs:
            d.wait()
        return _
    lax.fori_loop(0, HCH, hchunk_body, None)

    @pl.when(c == 1)
    def _():
        pltpu.sync_copy(idh.at[s], gi_v.at[pl.ds(0, 1)])
        pltpu.sync_copy(rows_v.at[1], acc_sp.at[gi_v.at[0]], add=True)

    plsc.subcore_barrier()
    _write_acc(s, rows_v, acc_sp, histH.at[c])
    _zero_acc(s, z128, rows_v, acc_sp)
    plsc.subcore_barrier()

    _ring_edges(tableB, gidxB, sidxB, c, s, gi_v, si_v, rows_v, acc_sp,
                (sg0, sg1))

    plsc.subcore_barrier()
    _write_acc(s, rows_v, acc_sp, aggB.at[c])


def _sc_pass_a_body(tableA, gidxA, sidxA, dgidx, tableG, tableBt, z128,
                    aggA, gammaM, betaM,
                    acc_sp, gi_v, si_v, rows_v, sg0, sg1):
    c = lax.axis_index("c")
    s = lax.axis_index("s")
    _zero_acc(s, z128, rows_v, acc_sp)
    plsc.subcore_barrier()

    _ring_edges(tableA, gidxA, sidxA, c, s, gi_v, si_v, rows_v, acc_sp,
                (sg0, sg1))

    pltpu.sync_copy(dgidx.at[c, s], gi_v.at[pl.ds(0, GB)])

    for gb in range(GB):
        ix = gi_v.at[gb]
        dg = pltpu.async_copy(tableG.at[ix], rows_v.at[0], sg0)
        db = pltpu.async_copy(tableBt.at[ix], rows_v.at[1], sg1)
        dg.wait()
        pltpu.sync_copy(rows_v.at[0],
                        gammaM.at[c, pl.ds(s * 640 + gb * EB, EB)])
        db.wait()
        pltpu.sync_copy(rows_v.at[1],
                        betaM.at[c, pl.ds(s * 640 + gb * EB, EB)])

    plsc.subcore_barrier()
    _write_acc(s, rows_v, acc_sp, aggA.at[c])


_SC_MESH = plsc.VectorSubcoreMesh(core_axis_name="c", subcore_axis_name="s")

_SC_SCRATCH = [
    pltpu.VMEM_SHARED((NACC, HC), jnp.float32),
    pltpu.VMEM((CH, EB), jnp.int32),
    pltpu.VMEM((CH, EB), jnp.int32),
    pltpu.VMEM((NBUF, EB, HC), jnp.float32),
    pltpu.SemaphoreType.DMA,
    pltpu.SemaphoreType.DMA,
]

_sc_pass_b = functools.partial(
    pl.kernel,
    _sc_pass_b_body,
    out_type=[
        jax.ShapeDtypeStruct((NC, NACC, HC), jnp.float32),
        jax.ShapeDtypeStruct((NC, NACC, HC), jnp.float32),
    ],
    mesh=_SC_MESH,
    scratch_types=_SC_SCRATCH,
)()

_sc_pass_a = functools.partial(
    pl.kernel,
    _sc_pass_a_body,
    out_type=[
        jax.ShapeDtypeStruct((NC, NACC, HC), jnp.float32),
        jax.ShapeDtypeStruct((NC, NPG, HC), jnp.float32),
        jax.ShapeDtypeStruct((NC, NPG, HC), jnp.float32),
    ],
    mesh=_SC_MESH,
    scratch_types=_SC_SCRATCH,
)()


BM = 1000


def _prep_body(x_ref, w_ref, pe_ref, wg_ref, wb_ref, bg_ref, bb_ref,
               degp_ref, h_ref, g_ref, bt_ref, sd_ref):
    i = pl.program_id(0)
    h_ref[...] = jnp.dot(x_ref[...], w_ref[...],
                         preferred_element_type=jnp.float32)

    @pl.when(i == 0)
    def _():
        g_ref[...] = jnp.dot(pe_ref[...], wg_ref[...],
                             preferred_element_type=jnp.float32) + bg_ref[...]
        bt_ref[...] = jnp.dot(pe_ref[...], wb_ref[...],
                              preferred_element_type=jnp.float32) + bb_ref[...]
        sd_ref[...] = jnp.sum(degp_ref[...]).reshape(1, 1)


def _mid_body(histH_ref, h_ref, hs_ref):
    deg_sl = histH_ref[0, :, 0:1] + histH_ref[1, :, 0:1] + 1.0
    hs_ref[...] = h_ref[...] * lax.rsqrt(deg_sl)


def _leaky(v):
    return jnp.where(v >= 0.0, v, 0.01 * v)


def _final_body(aggA_ref, aggB_ref, h_ref, histH_ref,
                gM_ref, bM_ref, deg_ref, wcat_ref, sd_ref,
                out_ref, lb_ref, lf_ref):
    i = pl.program_id(0)
    gamma = _leaky(jnp.concatenate([gM_ref[0], gM_ref[1]], axis=-1))
    beta = _leaky(jnp.concatenate([bM_ref[0], bM_ref[1]], axis=-1))
    deg = deg_ref[...]
    dis = lax.rsqrt(histH_ref[0, :, 0:1] + histH_ref[1, :, 0:1] + 1.0)
    A = jnp.concatenate([aggA_ref[0], aggA_ref[1]], axis=-1)
    conv_out = dis * A + (dis * dis) * h_ref[...]
    B8 = jnp.concatenate([aggB_ref[0], aggB_ref[1]], axis=-1) * 8.0
    zerod = deg == 0.0
    i_n = jnp.where(zerod, 0.0, B8 / jnp.where(zerod, 1.0, deg))
    t = jnp.dot(i_n, wcat_ref[...], preferred_element_type=jnp.float32)
    b_add = (gamma + 1.0) * t[:, :C] + beta
    b_rev = (gamma + 1.0) * t[:, C:] + beta
    Kv = sd_ref[...][0, 0] * (1.0 / N)
    R = (deg < Kv).astype(jnp.float32)
    out_ref[...] = conv_out + OMEGA * (R * b_add - (1.0 - R) * b_rev)
    na = jnp.sqrt(jnp.sum(b_add * b_add, axis=1, keepdims=True))
    nr = jnp.sqrt(jnp.sum(b_rev * b_rev, axis=1, keepdims=True))
    ng = jnp.sqrt(jnp.sum(gamma * gamma, axis=1, keepdims=True))
    nbt = jnp.sqrt(jnp.sum(beta * beta, axis=1, keepdims=True))
    cvec = histH_ref[0, :, 64:65] + histH_ref[1, :, 64:65]
    lb_p = jnp.sum(cvec * (R * na + (1.0 - R) * nr)) * (1.0 / B_IDX)
    lf_p = jnp.sum(cvec * (ng + nbt)) * (1.0 / B_IDX)

    @pl.when(i == 0)
    def _():
        lb_ref[...] = jnp.zeros((1, 1), jnp.float32)
        lf_ref[...] = jnp.zeros((1, 1), jnp.float32)

    lb_ref[...] += lb_p.reshape(1, 1)
    lf_ref[...] += lf_p.reshape(1, 1)


def kernel(x, adj, degree, idx, edge, W_conv, W_gamma, W_beta, b_gamma,
           b_beta, W_add, W_rev, PE):
    src = adj[0]
    dst = adj[1]
    f32 = jnp.float32

    padn = E_PAD - E
    srcp = jnp.concatenate([src, jnp.full((padn,), JUNK, jnp.int32)])
    dstp_s = jnp.concatenate([dst, jnp.full((padn,), JUNK, jnp.int32)])
    src0 = jnp.concatenate([src, jnp.zeros((padn,), jnp.int32)])
    dst0 = jnp.concatenate([dst, jnp.zeros((padn,), jnp.int32)])
    gidxB = jnp.stack([2 * dst0, 2 * dst0 + 1]).reshape(NC, NS, NCH, CH, EB)
    sidxB = srcp.reshape(NS, NCH, CH, EB)
    hidxB = dstp_s.reshape(NS, NCH, CH, EB)
    gidxA = jnp.stack([2 * src0, 2 * src0 + 1]).reshape(NC, NS, NCH, CH, EB)
    sidxA = dstp_s.reshape(NS, NCH, CH, EB)
    idh = idx.reshape(NS, 1, EB)
    deg0 = jnp.concatenate([degree[:, 0], jnp.zeros((NPG - N,), jnp.int32)])
    dgidx = jnp.stack([2 * deg0, 2 * deg0 + 1]).reshape(NC, NS, GB, EB)
    z128 = jnp.zeros((EB, HC), f32)
    col = jnp.arange(HC)[None, :]
    omask = jnp.stack([jnp.where(col < 64, 1.0, 0.0) * jnp.ones((EB, 1)),
                       jnp.where(col >= 64, 1.0, 0.0) * jnp.ones((EB, 1))],
                      axis=0).astype(f32)
    deg_f = degree.astype(f32)
    degp = jnp.concatenate([deg_f[:, 0], jnp.zeros((NPG - N,), f32)])
    degp = degp.reshape(80, 128)
    PEp = jnp.concatenate([PE, jnp.zeros((D_PAD - D_MAX, DIM_M), f32)])

    h, G, Bt, sd = pl.pallas_call(
        _prep_body,
        grid=(N // BM,),
        in_specs=[
            pl.BlockSpec((BM, C), lambda i: (i, 0)),
            pl.BlockSpec((C, C), lambda i: (0, 0)),
            pl.BlockSpec((D_PAD, DIM_M), lambda i: (0, 0)),
            pl.BlockSpec((DIM_M, C), lambda i: (0, 0)),
            pl.BlockSpec((DIM_M, C), lambda i: (0, 0)),
            pl.BlockSpec((1, C), lambda i: (0, 0)),
            pl.BlockSpec((1, C), lambda i: (0, 0)),
            pl.BlockSpec((80, 128), lambda i: (0, 0)),
        ],
        out_specs=[
            pl.BlockSpec((BM, C), lambda i: (i, 0)),
            pl.BlockSpec((D_PAD, C), lambda i: (0, 0)),
            pl.BlockSpec((D_PAD, C), lambda i: (0, 0)),
            pl.BlockSpec((1, 1), lambda i: (0, 0)),
        ],
        out_shape=[
            jax.ShapeDtypeStruct((N, C), f32),
            jax.ShapeDtypeStruct((D_PAD, C), f32),
            jax.ShapeDtypeStruct((D_PAD, C), f32),
            jax.ShapeDtypeStruct((1, 1), f32),
        ],
    )(x, W_conv, PEp, W_gamma, W_beta, b_gamma, b_beta, degp)

    tableB = h.reshape(2 * N, HC)
    aggB, histH = _sc_pass_b(tableB, gidxB, sidxB, hidxB, idh, z128, omask)

    hs = pl.pallas_call(
        _mid_body,
        grid=(N // BM,),
        in_specs=[
            pl.BlockSpec((NC, BM, HC), lambda i: (0, i, 0)),
            pl.BlockSpec((BM, C), lambda i: (i, 0)),
        ],
        out_specs=pl.BlockSpec((BM, C), lambda i: (i, 0)),
        out_shape=jax.ShapeDtypeStruct((N, C), f32),
    )(histH, h)

    tableA = hs.reshape(2 * N, HC)
    tableG = G.reshape(2 * D_PAD, HC)
    tableBt = Bt.reshape(2 * D_PAD, HC)
    aggA, gammaM, betaM = _sc_pass_a(tableA, gidxA, sidxA, dgidx,
                                     tableG, tableBt, z128)

    Wcat = jnp.concatenate([W_add, W_rev], axis=1)
    out, lb, lf = pl.pallas_call(
        _final_body,
        grid=(N // BM,),
        in_specs=[
            pl.BlockSpec((NC, BM, HC), lambda i: (0, i, 0)),
            pl.BlockSpec((NC, BM, HC), lambda i: (0, i, 0)),
            pl.BlockSpec((BM, C), lambda i: (i, 0)),
            pl.BlockSpec((NC, BM, HC), lambda i: (0, i, 0)),
            pl.BlockSpec((NC, BM, HC), lambda i: (0, i, 0)),
            pl.BlockSpec((NC, BM, HC), lambda i: (0, i, 0)),
            pl.BlockSpec((BM, 1), lambda i: (i, 0)),
            pl.BlockSpec((C, 2 * C), lambda i: (0, 0)),
            pl.BlockSpec((1, 1), lambda i: (0, 0)),
        ],
        out_specs=[
            pl.BlockSpec((BM, C), lambda i: (i, 0)),
            pl.BlockSpec((1, 1), lambda i: (0, 0)),
            pl.BlockSpec((1, 1), lambda i: (0, 0)),
        ],
        out_shape=[
            jax.ShapeDtypeStruct((N, C), f32),
            jax.ShapeDtypeStruct((1, 1), f32),
            jax.ShapeDtypeStruct((1, 1), f32),
        ],
    )(aggA, aggB, h, histH, gammaM, betaM, deg_f, Wcat, sd)

    return out, lb[0, 0], lf[0, 0]


# speedup vs baseline: 6.5714x; 1.0203x over previous
import functools

import jax
import jax.numpy as jnp
from jax import lax
from jax.experimental import pallas as pl
from jax.experimental.pallas import tpu as pltpu
from jax.experimental.pallas import tpu_sc as plsc

N = 10000
E = 160000
C = 256
HC = 128
DIM_M = 64
D_MAX = 612
OMEGA = 0.1
B_IDX = 2048

NC = 2
NS = 16
EB = 128
NB = 80
CH = 16
NCH = NB // CH
ET = EB * NB
E_PAD = ET * NS
JUNK = N
NACC = 10112
NPG = 10240
GB = 5
D_PAD = 616
NBUF = 2

CHUNKS_MAIN = ((0, 128), (128, 128), (256, 128), (384, 128), (512, 120))


def _stripe_copies(s, copy_fn):
    for off, sz in CHUNKS_MAIN:
        copy_fn(off, sz)


def _zero_acc(s, z128, rows_v, acc_sp):
    pltpu.sync_copy(z128, rows_v.at[0])

    def cp(off, sz):
        pltpu.sync_copy(rows_v.at[0, pl.ds(0, sz)],
                        acc_sp.at[pl.ds(s * 632 + off, sz)])
    _stripe_copies(s, cp)


def _write_acc(s, rows_v, acc_sp, out_ref):
    def cp(off, sz):
        pltpu.sync_copy(acc_sp.at[pl.ds(s * 632 + off, sz)],
                        rows_v.at[0, pl.ds(0, sz)])
        pltpu.sync_copy(rows_v.at[0, pl.ds(0, sz)],
                        out_ref.at[pl.ds(s * 632 + off, sz)])
    _stripe_copies(s, cp)


def _ring_edges(table, gidx, sidx, c, s, gi_v, si_v, rows_v, acc_sp,
                sg):
    def chunk_body(ch, _):
        pltpu.sync_copy(gidx.at[c, s, ch], gi_v)
        pltpu.sync_copy(sidx.at[s, ch], si_v)
        d = pltpu.async_copy(table.at[gi_v.at[0]], rows_v.at[0], sg[0])
        for b in range(CH):
            p = b % 2
            if b + 1 < CH:
                d_next = pltpu.async_copy(table.at[gi_v.at[b + 1]],
                                          rows_v.at[1 - p], sg[1 - p])
            d.wait()
            pltpu.sync_copy(rows_v.at[p], acc_sp.at[si_v.at[b]], add=True)
            if b + 1 < CH:
                d = d_next
        return _

    lax.fori_loop(0, NCH, chunk_body, None)


def _sc_pass_b_body(tableB, gidxB, sidxB, hidxB, idh, z128, omask,
                    aggB, histH,
                    acc_sp, gi_v, si_v, rows_v, sg0, sg1):
    c = lax.axis_index("c")
    s = lax.axis_index("s")

    _zero_acc(s, z128, rows_v, acc_sp)
    pltpu.sync_copy(omask.at[0], rows_v.at[0])
    pltpu.sync_copy(omask.at[1], rows_v.at[1])
    plsc.subcore_barrier()

    ones0 = rows_v.at[0]
    HCH = NCH // 2

    def hchunk_body(ch, _):
        pltpu.sync_copy(hidxB.at[s, ch + c * HCH], gi_v)
        descs = [pltpu.async_copy(ones0, acc_sp.at[gi_v.at[b]], sg0,
                                  add=True) for b in range(CH)]
        for d in descs:
            d.wait()
        return _
    lax.fori_loop(0, HCH, hchunk_body, None)

    @pl.when(c == 1)
    def _():
        pltpu.sync_copy(idh.at[s], gi_v.at[pl.ds(0, 1)])
        pltpu.sync_copy(rows_v.at[1], acc_sp.at[gi_v.at[0]], add=True)

    plsc.subcore_barrier()
    _write_acc(s, rows_v, acc_sp, histH.at[c])
    _zero_acc(s, z128, rows_v, acc_sp)
    plsc.subcore_barrier()

    _ring_edges(tableB, gidxB, sidxB, c, s, gi_v, si_v, rows_v, acc_sp,
                (sg0, sg1))

    plsc.subcore_barrier()
    _write_acc(s, rows_v, acc_sp, aggB.at[c])


def _sc_pass_a_body(tableA, gidxA, sidxA, dgidx, tableG, tableBt, z128,
                    aggA, gammaM, betaM,
                    acc_sp, gi_v, si_v, rows_v, sg0, sg1):
    c = lax.axis_index("c")
    s = lax.axis_index("s")
    _zero_acc(s, z128, rows_v, acc_sp)
    plsc.subcore_barrier()

    _ring_edges(tableA, gidxA, sidxA, c, s, gi_v, si_v, rows_v, acc_sp,
                (sg0, sg1))

    pltpu.sync_copy(dgidx.at[c, s], gi_v.at[pl.ds(0, GB)])

    for gb in range(GB):
        ix = gi_v.at[gb]
        dg = pltpu.async_copy(tableG.at[ix], rows_v.at[0], sg0)
        db = pltpu.async_copy(tableBt.at[ix], rows_v.at[1], sg1)
        dg.wait()
        pltpu.sync_copy(rows_v.at[0],
                        gammaM.at[c, pl.ds(s * 640 + gb * EB, EB)])
        db.wait()
        pltpu.sync_copy(rows_v.at[1],
                        betaM.at[c, pl.ds(s * 640 + gb * EB, EB)])

    plsc.subcore_barrier()
    _write_acc(s, rows_v, acc_sp, aggA.at[c])


_SC_MESH = plsc.VectorSubcoreMesh(core_axis_name="c", subcore_axis_name="s")

_SC_SCRATCH = [
    pltpu.VMEM_SHARED((NACC, HC), jnp.float32),
    pltpu.VMEM((CH, EB), jnp.int32),
    pltpu.VMEM((CH, EB), jnp.int32),
    pltpu.VMEM((NBUF, EB, HC), jnp.float32),
    pltpu.SemaphoreType.DMA,
    pltpu.SemaphoreType.DMA,
]

_sc_pass_b = functools.partial(
    pl.kernel,
    _sc_pass_b_body,
    out_type=[
        jax.ShapeDtypeStruct((NC, NACC, HC), jnp.float32),
        jax.ShapeDtypeStruct((NC, NACC, HC), jnp.float32),
    ],
    mesh=_SC_MESH,
    scratch_types=_SC_SCRATCH,
)()

_sc_pass_a = functools.partial(
    pl.kernel,
    _sc_pass_a_body,
    out_type=[
        jax.ShapeDtypeStruct((NC, NACC, HC), jnp.float32),
        jax.ShapeDtypeStruct((NC, NPG, HC), jnp.float32),
        jax.ShapeDtypeStruct((NC, NPG, HC), jnp.float32),
    ],
    mesh=_SC_MESH,
    scratch_types=_SC_SCRATCH,
)()


BM = 1000


def _prep_body(x_ref, w_ref, pe_ref, wg_ref, wb_ref, bg_ref, bb_ref,
               degp_ref, h_ref, g_ref, bt_ref, sd_ref):
    i = pl.program_id(0)
    h_ref[...] = jnp.dot(x_ref[...], w_ref[...],
                         preferred_element_type=jnp.float32)

    @pl.when(i == 0)
    def _():
        g_ref[...] = jnp.dot(pe_ref[...], wg_ref[...],
                             preferred_element_type=jnp.float32) + bg_ref[...]
        bt_ref[...] = jnp.dot(pe_ref[...], wb_ref[...],
                              preferred_element_type=jnp.float32) + bb_ref[...]
        sd_ref[...] = jnp.sum(degp_ref[...]).reshape(1, 1)


def _mid_body(histH_ref, h_ref, hs_ref):
    deg_sl = histH_ref[0, :, 0:1] + histH_ref[1, :, 0:1] + 1.0
    hs_ref[...] = h_ref[...] * lax.rsqrt(deg_sl)


def _leaky(v):
    return jnp.where(v >= 0.0, v, 0.01 * v)


def _final_body(aggA_ref, aggB_ref, h_ref, histH_ref,
                gM_ref, bM_ref, deg_ref, wcat_ref, sd_ref,
                out_ref, lb_ref, lf_ref):
    i = pl.program_id(0)
    gamma = _leaky(jnp.concatenate([gM_ref[0], gM_ref[1]], axis=-1))
    beta = _leaky(jnp.concatenate([bM_ref[0], bM_ref[1]], axis=-1))
    deg = deg_ref[...]
    dis = lax.rsqrt(histH_ref[0, :, 0:1] + histH_ref[1, :, 0:1] + 1.0)
    A = jnp.concatenate([aggA_ref[0], aggA_ref[1]], axis=-1)
    conv_out = dis * A + (dis * dis) * h_ref[...]
    B8 = jnp.concatenate([aggB_ref[0], aggB_ref[1]], axis=-1) * 8.0
    zerod = deg == 0.0
    i_n = jnp.where(zerod, 0.0, B8 / jnp.where(zerod, 1.0, deg))
    t = jnp.dot(i_n, wcat_ref[...], preferred_element_type=jnp.float32)
    b_add = (gamma + 1.0) * t[:, :C] + beta
    b_rev = (gamma + 1.0) * t[:, C:] + beta
    Kv = sd_ref[...][0, 0] * (1.0 / N)
    R = (deg < Kv).astype(jnp.float32)
    out_ref[...] = conv_out + OMEGA * (R * b_add - (1.0 - R) * b_rev)
    na = jnp.sqrt(jnp.sum(b_add * b_add, axis=1, keepdims=True))
    nr = jnp.sqrt(jnp.sum(b_rev * b_rev, axis=1, keepdims=True))
    ng = jnp.sqrt(jnp.sum(gamma * gamma, axis=1, keepdims=True))
    nbt = jnp.sqrt(jnp.sum(beta * beta, axis=1, keepdims=True))
    cvec = histH_ref[0, :, 64:65] + histH_ref[1, :, 64:65]
    lb_p = jnp.sum(cvec * (R * na + (1.0 - R) * nr)) * (1.0 / B_IDX)
    lf_p = jnp.sum(cvec * (ng + nbt)) * (1.0 / B_IDX)

    @pl.when(i == 0)
    def _():
        lb_ref[...] = jnp.zeros((1, 1), jnp.float32)
        lf_ref[...] = jnp.zeros((1, 1), jnp.float32)

    lb_ref[...] += lb_p.reshape(1, 1)
    lf_ref[...] += lf_p.reshape(1, 1)


def kernel(x, adj, degree, idx, edge, W_conv, W_gamma, W_beta, b_gamma,
           b_beta, W_add, W_rev, PE):
    src = adj[0]
    dst = adj[1]
    f32 = jnp.float32

    padn = E_PAD - E
    srcp = jnp.concatenate([src, jnp.full((padn,), JUNK, jnp.int32)])
    dstp_s = jnp.concatenate([dst, jnp.full((padn,), JUNK, jnp.int32)])
    src0 = jnp.concatenate([src, jnp.zeros((padn,), jnp.int32)])
    dst0 = jnp.concatenate([dst, jnp.zeros((padn,), jnp.int32)])
    gidxB = jnp.stack([2 * dst0, 2 * dst0 + 1]).reshape(NC, NS, NCH, CH, EB)
    sidxB = srcp.reshape(NS, NCH, CH, EB)
    hidxB = dstp_s.reshape(NS, NCH, CH, EB)
    gidxA = jnp.stack([2 * src0, 2 * src0 + 1]).reshape(NC, NS, NCH, CH, EB)
    sidxA = dstp_s.reshape(NS, NCH, CH, EB)
    idh = idx.reshape(NS, 1, EB)
    deg0 = jnp.concatenate([degree[:, 0], jnp.zeros((NPG - N,), jnp.int32)])
    dgidx = jnp.stack([2 * deg0, 2 * deg0 + 1]).reshape(NC, NS, GB, EB)
    z128 = jnp.zeros((EB, HC), f32)
    col = jnp.arange(HC)[None, :]
    omask = jnp.stack([jnp.where(col < 64, 1.0, 0.0) * jnp.ones((EB, 1)),
                       jnp.where(col >= 64, 1.0, 0.0) * jnp.ones((EB, 1))],
                      axis=0).astype(f32)
    deg_f = degree.astype(f32)
    degp = jnp.concatenate([deg_f[:, 0], jnp.zeros((NPG - N,), f32)])
    degp = degp.reshape(80, 128)
    PEp = jnp.concatenate([PE, jnp.zeros((D_PAD - D_MAX, DIM_M), f32)])

    h, G, Bt, sd = pl.pallas_call(
        _prep_body,
        grid=(N // BM,),
        in_specs=[
            pl.BlockSpec((BM, C), lambda i: (i, 0)),
            pl.BlockSpec((C, C), lambda i: (0, 0)),
            pl.BlockSpec((D_PAD, DIM_M), lambda i: (0, 0)),
            pl.BlockSpec((DIM_M, C), lambda i: (0, 0)),
            pl.BlockSpec((DIM_M, C), lambda i: (0, 0)),
            pl.BlockSpec((1, C), lambda i: (0, 0)),
            pl.BlockSpec((1, C), lambda i: (0, 0)),
            pl.BlockSpec((80, 128), lambda i: (0, 0)),
        ],
        out_specs=[
            pl.BlockSpec((BM, C), lambda i: (i, 0)),
            pl.BlockSpec((D_PAD, C), lambda i: (0, 0)),
            pl.BlockSpec((D_PAD, C), lambda i: (0, 0)),
            pl.BlockSpec((1, 1), lambda i: (0, 0)),
        ],
        out_shape=[
            jax.ShapeDtypeStruct((N, C), f32),
            jax.ShapeDtypeStruct((D_PAD, C), f32),
            jax.ShapeDtypeStruct((D_PAD, C), f32),
            jax.ShapeDtypeStruct((1, 1), f32),
        ],
    )(x, W_conv, PEp, W_gamma, W_beta, b_gamma, b_beta, degp)

    tableB = h.reshape(2 * N, HC)
    aggB, histH = _sc_pass_b(tableB, gidxB, sidxB, hidxB, idh, z128, omask)

    hs = pl.pallas_call(
        _mid_body,
        grid=(N // BM,),
        in_specs=[
            pl.BlockSpec((NC, BM, HC), lambda i: (0, i, 0)),
            pl.BlockSpec((BM, C), lambda i: (i, 0)),
        ],
        out_specs=pl.BlockSpec((BM, C), lambda i: (i, 0)),
        out_shape=jax.ShapeDtypeStruct((N, C), f32),
    )(histH, h)

    tableA = hs.reshape(2 * N, HC)
    tableG = G.reshape(2 * D_PAD, HC)
    tableBt = Bt.reshape(2 * D_PAD, HC)
    aggA, gammaM, betaM = _sc_pass_a(tableA, gidxA, sidxA, dgidx,
                                     tableG, tableBt, z128)

    Wcat = jnp.concatenate([W_add, W_rev], axis=1)
    out, lb, lf = pl.pallas_call(
        _final_body,
        grid=(N // BM,),
        in_specs=[
            pl.BlockSpec((NC, BM, HC), lambda i: (0, i, 0)),
            pl.BlockSpec((NC, BM, HC), lambda i: (0, i, 0)),
            pl.BlockSpec((BM, C), lambda i: (i, 0)),
            pl.BlockSpec((NC, BM, HC), lambda i: (0, i, 0)),
            pl.BlockSpec((NC, BM, HC), lambda i: (0, i, 0)),
            pl.BlockSpec((NC, BM, HC), lambda i: (0, i, 0)),
            pl.BlockSpec((BM, 1), lambda i: (i, 0)),
            pl.BlockSpec((C, 2 * C), lambda i: (0, 0)),
            pl.BlockSpec((1, 1), lambda i: (0, 0)),
        ],
        out_specs=[
            pl.BlockSpec((BM, C), lambda i: (i, 0)),
            pl.BlockSpec((1, 1), lambda i: (0, 0)),
            pl.BlockSpec((1, 1), lambda i: (0, 0)),
        ],
        out_shape=[
            jax.ShapeDtypeStruct((N, C), f32),
            jax.ShapeDtypeStruct((1, 1), f32),
            jax.ShapeDtypeStruct((1, 1), f32),
        ],
    )(aggA, aggB, h, histH, gammaM, betaM, deg_f, Wcat, sd)

    return out, lb[0, 0], lf[0, 0]


# speedup vs baseline: 6.5775x; 1.0009x over previous
import functools

import jax
import jax.numpy as jnp
from jax import lax
from jax.experimental import pallas as pl
from jax.experimental.pallas import tpu as pltpu
from jax.experimental.pallas import tpu_sc as plsc

N = 10000
E = 160000
C = 256
HC = 128
DIM_M = 64
D_MAX = 612
OMEGA = 0.1
B_IDX = 2048

NC = 2
NS = 16
EB = 128
NB = 80
CH = 16
NCH = NB // CH
ET = EB * NB
E_PAD = ET * NS
JUNK = N
NACC = 10112
NPG = 10240
GB = 5
D_PAD = 616
NBUF = 2

CHUNKS_MAIN = ((0, 128), (128, 128), (256, 128), (384, 128), (512, 120))


def _stripe_copies(s, copy_fn):
    for off, sz in CHUNKS_MAIN:
        copy_fn(off, sz)


def _zero_acc(s, z128, rows_v, acc_sp):
    pltpu.sync_copy(z128, rows_v.at[0])

    def cp(off, sz):
        pltpu.sync_copy(rows_v.at[0, pl.ds(0, sz)],
                        acc_sp.at[pl.ds(s * 632 + off, sz)])
    _stripe_copies(s, cp)


def _write_acc(s, rows_v, acc_sp, out_ref):
    def cp(off, sz):
        pltpu.sync_copy(acc_sp.at[pl.ds(s * 632 + off, sz)],
                        rows_v.at[0, pl.ds(0, sz)])
        pltpu.sync_copy(rows_v.at[0, pl.ds(0, sz)],
                        out_ref.at[pl.ds(s * 632 + off, sz)])
    _stripe_copies(s, cp)


def _ring_edges(table, gidx, sidx, c, s, gi_v, si_v, rows_v, acc_sp,
                sg):
    def chunk_body(ch, _):
        pltpu.sync_copy(gidx.at[c, s, ch], gi_v)
        pltpu.sync_copy(sidx.at[s, ch], si_v)
        d = pltpu.async_copy(table.at[gi_v.at[0]], rows_v.at[0], sg[0])
        for b in range(CH):
            p = b % 2
            if b + 1 < CH:
                d_next = pltpu.async_copy(table.at[gi_v.at[b + 1]],
                                          rows_v.at[1 - p], sg[1 - p])
            d.wait()
            pltpu.sync_copy(rows_v.at[p], acc_sp.at[si_v.at[b]], add=True)
            if b + 1 < CH:
                d = d_next
        return _

    lax.fori_loop(0, NCH, chunk_body, None)


def _sc_pass_b_body(tableB, gidxB, sidxB, hidxB, idh, z128, omask,
                    aggB, histH,
                    acc_sp, gi_v, si_v, rows_v, sg0, sg1):
    c = lax.axis_index("c")
    s = lax.axis_index("s")

    _zero_acc(s, z128, rows_v, acc_sp)
    pltpu.sync_copy(omask.at[0], rows_v.at[0])
    pltpu.sync_copy(omask.at[1], rows_v.at[1])
    plsc.subcore_barrier()

    ones0 = rows_v.at[0]
    HCH = NCH // 2

    def hchunk_body(ch, _):
        pltpu.sync_copy(hidxB.at[s, ch + c * HCH], gi_v)
        descs = [pltpu.async_copy(ones0, acc_sp.at[gi_v.at[b]], sg0,
                                  add=True) for b in range(CH)]
        for d in descs:
            d.wait()
        return _
    lax.fori_loop(0, HCH, hchunk_body, None)

    if NCH % 2 == 1:
        @pl.when(c == 0)
        def _():
            hchunk_body(NCH - 1, None)

    @pl.when(c == 1)
    def _():
        pltpu.sync_copy(idh.at[s], gi_v.at[pl.ds(0, 1)])
        pltpu.sync_copy(rows_v.at[1], acc_sp.at[gi_v.at[0]], add=True)

    plsc.subcore_barrier()
    _write_acc(s, rows_v, acc_sp, histH.at[c])
    _zero_acc(s, z128, rows_v, acc_sp)
    plsc.subcore_barrier()

    _ring_edges(tableB, gidxB, sidxB, c, s, gi_v, si_v, rows_v, acc_sp,
                (sg0, sg1))

    plsc.subcore_barrier()
    _write_acc(s, rows_v, acc_sp, aggB.at[c])


def _sc_pass_a_body(tableA, gidxA, sidxA, dgidx, tableG, tableBt, z128,
                    aggA, gammaM, betaM,
                    acc_sp, gi_v, si_v, rows_v, sg0, sg1):
    c = lax.axis_index("c")
    s = lax.axis_index("s")
    _zero_acc(s, z128, rows_v, acc_sp)
    plsc.subcore_barrier()

    _ring_edges(tableA, gidxA, sidxA, c, s, gi_v, si_v, rows_v, acc_sp,
                (sg0, sg1))

    pltpu.sync_copy(dgidx.at[c, s], gi_v.at[pl.ds(0, GB)])

    for gb in range(GB):
        ix = gi_v.at[gb]
        dg = pltpu.async_copy(tableG.at[ix], rows_v.at[0], sg0)
        db = pltpu.async_copy(tableBt.at[ix], rows_v.at[1], sg1)
        dg.wait()
        pltpu.sync_copy(rows_v.at[0],
                        gammaM.at[c, pl.ds(s * 640 + gb * EB, EB)])
        db.wait()
        pltpu.sync_copy(rows_v.at[1],
                        betaM.at[c, pl.ds(s * 640 + gb * EB, EB)])

    plsc.subcore_barrier()
    _write_acc(s, rows_v, acc_sp, aggA.at[c])


_SC_MESH = plsc.VectorSubcoreMesh(core_axis_name="c", subcore_axis_name="s")

_SC_SCRATCH = [
    pltpu.VMEM_SHARED((NACC, HC), jnp.float32),
    pltpu.VMEM((CH, EB), jnp.int32),
    pltpu.VMEM((CH, EB), jnp.int32),
    pltpu.VMEM((NBUF, EB, HC), jnp.float32),
    pltpu.SemaphoreType.DMA,
    pltpu.SemaphoreType.DMA,
]

_sc_pass_b = functools.partial(
    pl.kernel,
    _sc_pass_b_body,
    out_type=[
        jax.ShapeDtypeStruct((NC, NACC, HC), jnp.float32),
        jax.ShapeDtypeStruct((NC, NACC, HC), jnp.float32),
    ],
    mesh=_SC_MESH,
    scratch_types=_SC_SCRATCH,
)()

_sc_pass_a = functools.partial(
    pl.kernel,
    _sc_pass_a_body,
    out_type=[
        jax.ShapeDtypeStruct((NC, NACC, HC), jnp.float32),
        jax.ShapeDtypeStruct((NC, NPG, HC), jnp.float32),
        jax.ShapeDtypeStruct((NC, NPG, HC), jnp.float32),
    ],
    mesh=_SC_MESH,
    scratch_types=_SC_SCRATCH,
)()


BM = 1000


def _prep_body(x_ref, w_ref, pe_ref, wg_ref, wb_ref, bg_ref, bb_ref,
               degp_ref, h_ref, g_ref, bt_ref, sd_ref):
    i = pl.program_id(0)
    h_ref[...] = jnp.dot(x_ref[...], w_ref[...],
                         preferred_element_type=jnp.float32)

    @pl.when(i == 0)
    def _():
        g_ref[...] = jnp.dot(pe_ref[...], wg_ref[...],
                             preferred_element_type=jnp.float32) + bg_ref[...]
        bt_ref[...] = jnp.dot(pe_ref[...], wb_ref[...],
                              preferred_element_type=jnp.float32) + bb_ref[...]
        sd_ref[...] = jnp.sum(degp_ref[...]).reshape(1, 1)


def _mid_body(histH_ref, h_ref, hs_ref):
    deg_sl = histH_ref[0, :, 0:1] + histH_ref[1, :, 0:1] + 1.0
    hs_ref[...] = h_ref[...] * lax.rsqrt(deg_sl)


def _leaky(v):
    return jnp.where(v >= 0.0, v, 0.01 * v)


def _final_body(aggA_ref, aggB_ref, h_ref, histH_ref,
                gM_ref, bM_ref, deg_ref, wcat_ref, sd_ref,
                out_ref, lb_ref, lf_ref):
    i = pl.program_id(0)
    gamma = _leaky(jnp.concatenate([gM_ref[0], gM_ref[1]], axis=-1))
    beta = _leaky(jnp.concatenate([bM_ref[0], bM_ref[1]], axis=-1))
    deg = deg_ref[...]
    dis = lax.rsqrt(histH_ref[0, :, 0:1] + histH_ref[1, :, 0:1] + 1.0)
    A = jnp.concatenate([aggA_ref[0], aggA_ref[1]], axis=-1)
    conv_out = dis * A + (dis * dis) * h_ref[...]
    B8 = jnp.concatenate([aggB_ref[0], aggB_ref[1]], axis=-1) * 8.0
    zerod = deg == 0.0
    i_n = jnp.where(zerod, 0.0, B8 / jnp.where(zerod, 1.0, deg))
    t = jnp.dot(i_n, wcat_ref[...], preferred_element_type=jnp.float32)
    b_add = (gamma + 1.0) * t[:, :C] + beta
    b_rev = (gamma + 1.0) * t[:, C:] + beta
    Kv = sd_ref[...][0, 0] * (1.0 / N)
    R = (deg < Kv).astype(jnp.float32)
    out_ref[...] = conv_out + OMEGA * (R * b_add - (1.0 - R) * b_rev)
    na = jnp.sqrt(jnp.sum(b_add * b_add, axis=1, keepdims=True))
    nr = jnp.sqrt(jnp.sum(b_rev * b_rev, axis=1, keepdims=True))
    ng = jnp.sqrt(jnp.sum(gamma * gamma, axis=1, keepdims=True))
    nbt = jnp.sqrt(jnp.sum(beta * beta, axis=1, keepdims=True))
    cvec = histH_ref[0, :, 64:65] + histH_ref[1, :, 64:65]
    lb_p = jnp.sum(cvec * (R * na + (1.0 - R) * nr)) * (1.0 / B_IDX)
    lf_p = jnp.sum(cvec * (ng + nbt)) * (1.0 / B_IDX)

    @pl.when(i == 0)
    def _():
        lb_ref[...] = jnp.zeros((1, 1), jnp.float32)
        lf_ref[...] = jnp.zeros((1, 1), jnp.float32)

    lb_ref[...] += lb_p.reshape(1, 1)
    lf_ref[...] += lf_p.reshape(1, 1)


def kernel(x, adj, degree, idx, edge, W_conv, W_gamma, W_beta, b_gamma,
           b_beta, W_add, W_rev, PE):
    src = adj[0]
    dst = adj[1]
    f32 = jnp.float32

    padn = E_PAD - E
    srcp = jnp.concatenate([src, jnp.full((padn,), JUNK, jnp.int32)])
    dstp_s = jnp.concatenate([dst, jnp.full((padn,), JUNK, jnp.int32)])
    src0 = jnp.concatenate([src, jnp.zeros((padn,), jnp.int32)])
    dst0 = jnp.concatenate([dst, jnp.zeros((padn,), jnp.int32)])
    gidxB = jnp.stack([2 * dst0, 2 * dst0 + 1]).reshape(NC, NS, NCH, CH, EB)
    sidxB = srcp.reshape(NS, NCH, CH, EB)
    hidxB = dstp_s.reshape(NS, NCH, CH, EB)
    gidxA = jnp.stack([2 * src0, 2 * src0 + 1]).reshape(NC, NS, NCH, CH, EB)
    sidxA = dstp_s.reshape(NS, NCH, CH, EB)
    idh = idx.reshape(NS, 1, EB)
    deg0 = jnp.concatenate([degree[:, 0], jnp.zeros((NPG - N,), jnp.int32)])
    dgidx = jnp.stack([2 * deg0, 2 * deg0 + 1]).reshape(NC, NS, GB, EB)
    z128 = jnp.zeros((EB, HC), f32)
    col = jnp.arange(HC)[None, :]
    omask = jnp.stack([jnp.where(col < 64, 1.0, 0.0) * jnp.ones((EB, 1)),
                       jnp.where(col >= 64, 1.0, 0.0) * jnp.ones((EB, 1))],
                      axis=0).astype(f32)
    deg_f = degree.astype(f32)
    degp = jnp.concatenate([deg_f[:, 0], jnp.zeros((NPG - N,), f32)])
    degp = degp.reshape(80, 128)
    PEp = jnp.concatenate([PE, jnp.zeros((D_PAD - D_MAX, DIM_M), f32)])

    h, G, Bt, sd = pl.pallas_call(
        _prep_body,
        grid=(N // BM,),
        in_specs=[
            pl.BlockSpec((BM, C), lambda i: (i, 0)),
            pl.BlockSpec((C, C), lambda i: (0, 0)),
            pl.BlockSpec((D_PAD, DIM_M), lambda i: (0, 0)),
            pl.BlockSpec((DIM_M, C), lambda i: (0, 0)),
            pl.BlockSpec((DIM_M, C), lambda i: (0, 0)),
            pl.BlockSpec((1, C), lambda i: (0, 0)),
            pl.BlockSpec((1, C), lambda i: (0, 0)),
            pl.BlockSpec((80, 128), lambda i: (0, 0)),
        ],
        out_specs=[
            pl.BlockSpec((BM, C), lambda i: (i, 0)),
            pl.BlockSpec((D_PAD, C), lambda i: (0, 0)),
            pl.BlockSpec((D_PAD, C), lambda i: (0, 0)),
            pl.BlockSpec((1, 1), lambda i: (0, 0)),
        ],
        out_shape=[
            jax.ShapeDtypeStruct((N, C), f32),
            jax.ShapeDtypeStruct((D_PAD, C), f32),
            jax.ShapeDtypeStruct((D_PAD, C), f32),
            jax.ShapeDtypeStruct((1, 1), f32),
        ],
    )(x, W_conv, PEp, W_gamma, W_beta, b_gamma, b_beta, degp)

    tableB = h.reshape(2 * N, HC)
    aggB, histH = _sc_pass_b(tableB, gidxB, sidxB, hidxB, idh, z128, omask)

    hs = pl.pallas_call(
        _mid_body,
        grid=(N // BM,),
        in_specs=[
            pl.BlockSpec((NC, BM, HC), lambda i: (0, i, 0)),
            pl.BlockSpec((BM, C), lambda i: (i, 0)),
        ],
        out_specs=pl.BlockSpec((BM, C), lambda i: (i, 0)),
        out_shape=jax.ShapeDtypeStruct((N, C), f32),
    )(histH, h)

    tableA = hs.reshape(2 * N, HC)
    tableG = G.reshape(2 * D_PAD, HC)
    tableBt = Bt.reshape(2 * D_PAD, HC)
    aggA, gammaM, betaM = _sc_pass_a(tableA, gidxA, sidxA, dgidx,
                                     tableG, tableBt, z128)

    Wcat = jnp.concatenate([W_add, W_rev], axis=1)
    out, lb, lf = pl.pallas_call(
        _final_body,
        grid=(N // BM,),
        in_specs=[
            pl.BlockSpec((NC, BM, HC), lambda i: (0, i, 0)),
            pl.BlockSpec((NC, BM, HC), lambda i: (0, i, 0)),
            pl.BlockSpec((BM, C), lambda i: (i, 0)),
            pl.BlockSpec((NC, BM, HC), lambda i: (0, i, 0)),
            pl.BlockSpec((NC, BM, HC), lambda i: (0, i, 0)),
            pl.BlockSpec((NC, BM, HC), lambda i: (0, i, 0)),
            pl.BlockSpec((BM, 1), lambda i: (i, 0)),
            pl.BlockSpec((C, 2 * C), lambda i: (0, 0)),
            pl.BlockSpec((1, 1), lambda i: (0, 0)),
        ],
        out_specs=[
            pl.BlockSpec((BM, C), lambda i: (i, 0)),
            pl.BlockSpec((1, 1), lambda i: (0, 0)),
            pl.BlockSpec((1, 1), lambda i: (0, 0)),
        ],
        out_shape=[
            jax.ShapeDtypeStruct((N, C), f32),
            jax.ShapeDtypeStruct((1, 1), f32),
            jax.ShapeDtypeStruct((1, 1), f32),
        ],
    )(aggA, aggB, h, histH, gammaM, betaM, deg_f, Wcat, sd)

    return out, lb[0, 0], lf[0, 0]
